# baseline (speedup 1.0000x reference)
import functools

import jax
import jax.numpy as jnp
from jax import lax
from jax.experimental import pallas as pl
from jax.experimental.pallas import tpu as pltpu
from jax.experimental.pallas import tpu_sc as plsc

_B = 16384
_D = 128
_NC = 2
_NS = 16
_NW = _NC * _NS
_BPW = _B // _NW
_CH = 64
_NCHUNK = _BPW // _CH

_mesh = plsc.VectorSubcoreMesh(core_axis_name="c", subcore_axis_name="s")


@functools.partial(
    pl.kernel,
    mesh=_mesh,
    out_type=(
        jax.ShapeDtypeStruct((_B, 16), jnp.float32),
        jax.ShapeDtypeStruct((_B,), jnp.float32),
    ),
    scratch_types=[
        pltpu.VMEM((_CH,), jnp.int32),
        pltpu.VMEM((_CH,), jnp.int32),
        pltpu.VMEM((_CH,), jnp.int32),
        pltpu.VMEM((_CH,), jnp.int32),
        pltpu.VMEM((_CH,), jnp.int32),
        pltpu.VMEM((_CH,), jnp.int32),
        pltpu.VMEM((_CH, _D), jnp.float32),
        pltpu.VMEM((_CH, _D), jnp.float32),
        pltpu.VMEM((_CH, _D), jnp.float32),
        pltpu.VMEM((_CH, _D), jnp.float32),
        pltpu.VMEM((_CH, _D), jnp.float32),
        pltpu.VMEM((_CH, _D), jnp.float32),
        pltpu.VMEM((_CH,), jnp.float32),
        pltpu.VMEM((_CH,), jnp.float32),
        pltpu.VMEM((_CH,), jnp.float32),
        pltpu.VMEM((_CH,), jnp.float32),
        pltpu.VMEM((_CH, 16), jnp.float32),
        pltpu.VMEM((_CH,), jnp.float32),
        pltpu.SemaphoreType.DMA,
        pltpu.SemaphoreType.DMA,
    ],
)
def _cplr_sc(users, pos_items, neg_items, item_biases, ue, ie,
             part_out, bias_out,
             iu0, ip0, in0, iu1, ip1, in1,
             ru0, rp0, rn0, ru1, rp1, rn1,
             pb0, nb0, pb1, nb1, pvmem, bias_v, sem0, sem1):
    wid = lax.axis_index("s") * _NC + lax.axis_index("c")
    base = wid * _BPW

    idx_sets = [(iu0, ip0, in0), (iu1, ip1, in1)]
    row_sets = [(ru0, rp0, rn0), (ru1, rp1, rn1)]
    bias_sets = [(pb0, nb0), (pb1, nb1)]
    sems = [sem0, sem1]

    def stage_and_fire(c):
        s = c % 2
        iu, ip_, in_ = idx_sets[s]
        ru, rp, rn = row_sets[s]
        pb, nb = bias_sets[s]
        off = base + c * _CH
        pltpu.sync_copy(users.at[pl.ds(off, _CH)], iu)
        pltpu.sync_copy(pos_items.at[pl.ds(off, _CH)], ip_)
        pltpu.sync_copy(neg_items.at[pl.ds(off, _CH)], in_)
        return [
            pltpu.async_copy(ue.at[iu], ru, sems[s]),
            pltpu.async_copy(ie.at[ip_], rp, sems[s]),
            pltpu.async_copy(ie.at[in_], rn, sems[s]),
            pltpu.async_copy(item_biases.at[ip_], pb, sems[s]),
            pltpu.async_copy(item_biases.at[in_], nb, sems[s]),
        ]

    pending = {0: stage_and_fire(0)}
    for c in range(_NCHUNK):
        s = c % 2
        if c + 1 < _NCHUNK:
            pending[c + 1] = stage_and_fire(c + 1)
        for cp in pending.pop(c):
            cp.wait()
        ru, rp, rn = row_sets[s]
        pb, nb = bias_sets[s]
        off = base + c * _CH

        def group_body(g, carry, ru=ru, rp=rp, rn=rn, pb=pb, nb=nb):
            e0 = g * 16
            for i in range(16):
                e = e0 + i
                prods = []
                for j in range(_D // 16):
                    u = ru[e, pl.ds(j * 16, 16)]
                    p = rp[e, pl.ds(j * 16, 16)]
                    n = rn[e, pl.ds(j * 16, 16)]
                    prods.append(u * (p - n))
                while len(prods) > 1:
                    prods = [prods[k] + prods[k + 1]
                             for k in range(0, len(prods), 2)]
                pvmem[e, :] = prods[0]
            bias_v[pl.ds(e0, 16)] = pb[pl.ds(e0, 16)] - nb[pl.ds(e0, 16)]
            return carry

        lax.fori_loop(0, _CH // 16, group_body, 0)
        pltpu.sync_copy(pvmem, part_out.at[pl.ds(off, _CH), :])
        pltpu.sync_copy(bias_v, bias_out.at[pl.ds(off, _CH)])


def _reduce_tc_body(part_ref, bias_ref, out_ref):
    out_ref[...] = jnp.sum(part_ref[...], axis=-1) + bias_ref[...]


_reduce_tc = pl.pallas_call(
    _reduce_tc_body,
    out_shape=jax.ShapeDtypeStruct((_B,), jnp.float32),
    grid=(8,),
    in_specs=[
        pl.BlockSpec((_B // 8, 16), lambda i: (i, 0)),
        pl.BlockSpec((_B // 8,), lambda i: (i,)),
    ],
    out_specs=pl.BlockSpec((_B // 8,), lambda i: (i,)),
)


def kernel(users, pos_items, neg_items, user_biases, item_biases,
           user_embeddings, item_embeddings):
    del user_biases
    partials, bias_diff = _cplr_sc(
        users.astype(jnp.int32),
        pos_items.astype(jnp.int32),
        neg_items.astype(jnp.int32),
        item_biases.reshape(-1),
        user_embeddings,
        item_embeddings,
    )
    return _reduce_tc(partials, bias_diff)


# speedup vs baseline: 1.1129x; 1.1129x over previous
import functools

import jax
import jax.numpy as jnp
from jax import lax
from jax.experimental import pallas as pl
from jax.experimental.pallas import tpu as pltpu
from jax.experimental.pallas import tpu_sc as plsc

_B = 16384
_D = 128
_NC = 2
_NS = 16
_NW = _NC * _NS
_BPW = _B // _NW
_CH = 64
_NCHUNK = _BPW // _CH

_mesh = plsc.VectorSubcoreMesh(core_axis_name="c", subcore_axis_name="s")


@functools.partial(
    pl.kernel,
    mesh=_mesh,
    out_type=(
        jax.ShapeDtypeStruct((_B * 16,), jnp.float32),
        jax.ShapeDtypeStruct((_B,), jnp.float32),
    ),
    scratch_types=[
        pltpu.VMEM((_CH,), jnp.int32),
        pltpu.VMEM((_CH,), jnp.int32),
        pltpu.VMEM((_CH,), jnp.int32),
        pltpu.VMEM((_CH,), jnp.int32),
        pltpu.VMEM((_CH,), jnp.int32),
        pltpu.VMEM((_CH,), jnp.int32),
        pltpu.VMEM((_CH, _D), jnp.float32),
        pltpu.VMEM((_CH, _D), jnp.float32),
        pltpu.VMEM((_CH, _D), jnp.float32),
        pltpu.VMEM((_CH, _D), jnp.float32),
        pltpu.VMEM((_CH, _D), jnp.float32),
        pltpu.VMEM((_CH, _D), jnp.float32),
        pltpu.VMEM((_CH,), jnp.float32),
        pltpu.VMEM((_CH,), jnp.float32),
        pltpu.VMEM((_CH,), jnp.float32),
        pltpu.VMEM((_CH,), jnp.float32),
        pltpu.VMEM((_CH * 16,), jnp.float32),
        pltpu.VMEM((_CH,), jnp.float32),
        pltpu.SemaphoreType.DMA,
        pltpu.SemaphoreType.DMA,
    ],
)
def _cplr_sc(users, pos_items, neg_items, item_biases, ue, ie,
             part_out, bias_out,
             iu0, ip0, in0, iu1, ip1, in1,
             ru0, rp0, rn0, ru1, rp1, rn1,
             pb0, nb0, pb1, nb1, pvmem, bias_v, sem0, sem1):
    wid = lax.axis_index("s") * _NC + lax.axis_index("c")
    base = wid * _BPW

    idx_sets = [(iu0, ip0, in0), (iu1, ip1, in1)]
    row_sets = [(ru0, rp0, rn0), (ru1, rp1, rn1)]
    bias_sets = [(pb0, nb0), (pb1, nb1)]
    sems = [sem0, sem1]

    def stage_and_fire(c):
        s = c % 2
        iu, ip_, in_ = idx_sets[s]
        ru, rp, rn = row_sets[s]
        pb, nb = bias_sets[s]
        off = base + c * _CH
        pltpu.sync_copy(users.at[pl.ds(off, _CH)], iu)
        pltpu.sync_copy(pos_items.at[pl.ds(off, _CH)], ip_)
        pltpu.sync_copy(neg_items.at[pl.ds(off, _CH)], in_)
        return [
            pltpu.async_copy(ue.at[iu], ru, sems[s]),
            pltpu.async_copy(ie.at[ip_], rp, sems[s]),
            pltpu.async_copy(ie.at[in_], rn, sems[s]),
            pltpu.async_copy(item_biases.at[ip_], pb, sems[s]),
            pltpu.async_copy(item_biases.at[in_], nb, sems[s]),
        ]

    pending = {0: stage_and_fire(0)}
    for c in range(_NCHUNK):
        s = c % 2
        if c + 1 < _NCHUNK:
            pending[c + 1] = stage_and_fire(c + 1)
        for cp in pending.pop(c):
            cp.wait()
        ru, rp, rn = row_sets[s]
        pb, nb = bias_sets[s]
        off = base + c * _CH

        def group_body(g, carry, ru=ru, rp=rp, rn=rn, pb=pb, nb=nb):
            e0 = g * 16
            for i in range(16):
                e = e0 + i
                prods = []
                for j in range(_D // 16):
                    u = ru[e, pl.ds(j * 16, 16)]
                    p = rp[e, pl.ds(j * 16, 16)]
                    n = rn[e, pl.ds(j * 16, 16)]
                    prods.append(u * (p - n))
                while len(prods) > 1:
                    prods = [prods[k] + prods[k + 1]
                             for k in range(0, len(prods), 2)]
                pvmem[pl.ds(16 * e, 16)] = prods[0]
            bias_v[pl.ds(e0, 16)] = pb[pl.ds(e0, 16)] - nb[pl.ds(e0, 16)]
            return carry

        lax.fori_loop(0, _CH // 16, group_body, 0)
        pltpu.sync_copy(pvmem, part_out.at[pl.ds(16 * off, 16 * _CH)])
        pltpu.sync_copy(bias_v, bias_out.at[pl.ds(off, _CH)])


def _reduce_tc_body(part_ref, bias_ref, out_ref):
    out_ref[...] = jnp.sum(part_ref[...], axis=-1) + bias_ref[...]


_reduce_tc = pl.pallas_call(
    _reduce_tc_body,
    out_shape=jax.ShapeDtypeStruct((_B,), jnp.float32),
    grid=(8,),
    in_specs=[
        pl.BlockSpec((_B // 8, 16), lambda i: (i, 0)),
        pl.BlockSpec((_B // 8,), lambda i: (i,)),
    ],
    out_specs=pl.BlockSpec((_B // 8,), lambda i: (i,)),
)


def kernel(users, pos_items, neg_items, user_biases, item_biases,
           user_embeddings, item_embeddings):
    del user_biases
    partials, bias_diff = _cplr_sc(
        users.astype(jnp.int32),
        pos_items.astype(jnp.int32),
        neg_items.astype(jnp.int32),
        item_biases.reshape(-1),
        user_embeddings,
        item_embeddings,
    )
    return _reduce_tc(partials.reshape(_B, 16), bias_diff)


# speedup vs baseline: 1.7666x; 1.5873x over previous
import functools

import jax
import jax.numpy as jnp
from jax import lax
from jax.experimental import pallas as pl
from jax.experimental.pallas import tpu as pltpu
from jax.experimental.pallas import tpu_sc as plsc

_B = 16384
_D = 128
_NC = 2
_NS = 16
_NW = _NC * _NS
_BPW = _B // _NW
_CH = 128
_NCHUNK = _BPW // _CH

_mesh = plsc.VectorSubcoreMesh(core_axis_name="c", subcore_axis_name="s")


@functools.partial(
    pl.kernel,
    mesh=_mesh,
    out_type=jax.ShapeDtypeStruct((_B,), jnp.float32),
    scratch_types=[
        pltpu.VMEM((_CH,), jnp.int32),
        pltpu.VMEM((_CH,), jnp.int32),
        pltpu.VMEM((_CH,), jnp.int32),
        pltpu.VMEM((_CH,), jnp.int32),
        pltpu.VMEM((_CH,), jnp.int32),
        pltpu.VMEM((_CH,), jnp.int32),
        pltpu.VMEM((_CH, _D), jnp.float32),
        pltpu.VMEM((_CH, _D), jnp.float32),
        pltpu.VMEM((_CH, _D), jnp.float32),
        pltpu.VMEM((_CH, _D), jnp.float32),
        pltpu.VMEM((_CH, _D), jnp.float32),
        pltpu.VMEM((_CH, _D), jnp.float32),
        pltpu.VMEM((_CH,), jnp.float32),
        pltpu.VMEM((_CH,), jnp.float32),
        pltpu.VMEM((_CH,), jnp.float32),
        pltpu.VMEM((_CH,), jnp.float32),
        pltpu.VMEM((_CH,), jnp.float32),
        pltpu.SemaphoreType.DMA,
        pltpu.SemaphoreType.DMA,
    ],
)
def _cplr_sc(users, pos_items, neg_items, item_biases, ue, ie, out,
             iu0, ip0, in0, iu1, ip1, in1,
             ru0, rp0, rn0, ru1, rp1, rn1,
             pb0, nb0, pb1, nb1, out_v, sem0, sem1):
    wid = lax.axis_index("s") * _NC + lax.axis_index("c")
    base = wid * _BPW
    lanes = lax.iota(jnp.int32, 16)

    idx_sets = [(iu0, ip0, in0), (iu1, ip1, in1)]
    row_sets = [(ru0, rp0, rn0), (ru1, rp1, rn1)]
    bias_sets = [(pb0, nb0), (pb1, nb1)]
    sems = [sem0, sem1]

    def stage_and_fire(c):
        s = c % 2
        iu, ip_, in_ = idx_sets[s]
        ru, rp, rn = row_sets[s]
        pb, nb = bias_sets[s]
        off = base + c * _CH
        pltpu.sync_copy(users.at[pl.ds(off, _CH)], iu)
        pltpu.sync_copy(pos_items.at[pl.ds(off, _CH)], ip_)
        pltpu.sync_copy(neg_items.at[pl.ds(off, _CH)], in_)
        return [
            pltpu.async_copy(ue.at[iu], ru, sems[s]),
            pltpu.async_copy(ie.at[ip_], rp, sems[s]),
            pltpu.async_copy(ie.at[in_], rn, sems[s]),
            pltpu.async_copy(item_biases.at[ip_], pb, sems[s]),
            pltpu.async_copy(item_biases.at[in_], nb, sems[s]),
        ]

    pending = {0: stage_and_fire(0)}
    for c in range(_NCHUNK):
        s = c % 2
        if c + 1 < _NCHUNK:
            pending[c + 1] = stage_and_fire(c + 1)
        for cp in pending.pop(c):
            cp.wait()
        ru, rp, rn = row_sets[s]
        pb, nb = bias_sets[s]
        off = base + c * _CH

        @plsc.parallel_loop(0, _CH // 16, step=1)
        def group_body(g, ru=ru, rp=rp, rn=rn, pb=pb, nb=nb):
            e0 = g * 16
            tot = jnp.zeros((16,), jnp.float32)
            for i in range(16):
                e = e0 + i
                prods = []
                for j in range(_D // 16):
                    u = ru[e, pl.ds(j * 16, 16)]
                    p = rp[e, pl.ds(j * 16, 16)]
                    n = rn[e, pl.ds(j * 16, 16)]
                    prods.append(u * (p - n))
                while len(prods) > 1:
                    prods = [prods[k] + prods[k + 1]
                             for k in range(0, len(prods), 2)]
                acc = prods[0]
                t = acc[0]
                for k in range(1, 16):
                    t = t + acc[k]
                tot = jnp.where(lanes == i, t, tot)
            out_v[pl.ds(e0, 16)] = (tot + pb[pl.ds(e0, 16)]
                                    - nb[pl.ds(e0, 16)])

        pltpu.sync_copy(out_v, out.at[pl.ds(off, _CH)])


def kernel(users, pos_items, neg_items, user_biases, item_biases,
           user_embeddings, item_embeddings):
    del user_biases
    return _cplr_sc(
        users.astype(jnp.int32),
        pos_items.astype(jnp.int32),
        neg_items.astype(jnp.int32),
        item_biases.reshape(-1),
        user_embeddings,
        item_embeddings,
    )


# speedup vs baseline: 1.8426x; 1.0431x over previous
import functools

import jax
import jax.numpy as jnp
from jax import lax
from jax.experimental import pallas as pl
from jax.experimental.pallas import tpu as pltpu
from jax.experimental.pallas import tpu_sc as plsc

_B = 16384
_D = 128
_NC = 2
_NS = 16
_NW = _NC * _NS
_BPW = _B // _NW
_CH = 128
_NCHUNK = _BPW // _CH

_mesh = plsc.VectorSubcoreMesh(core_axis_name="c", subcore_axis_name="s")


@functools.partial(
    pl.kernel,
    mesh=_mesh,
    out_type=jax.ShapeDtypeStruct((_B,), jnp.float32),
    scratch_types=[
        pltpu.VMEM((_BPW,), jnp.int32),
        pltpu.VMEM((_BPW,), jnp.int32),
        pltpu.VMEM((_BPW,), jnp.int32),
        pltpu.VMEM((_CH, _D), jnp.float32),
        pltpu.VMEM((_CH, _D), jnp.float32),
        pltpu.VMEM((_CH, _D), jnp.float32),
        pltpu.VMEM((_CH, _D), jnp.float32),
        pltpu.VMEM((_CH, _D), jnp.float32),
        pltpu.VMEM((_CH, _D), jnp.float32),
        pltpu.VMEM((_CH,), jnp.float32),
        pltpu.VMEM((_CH,), jnp.float32),
        pltpu.VMEM((_CH,), jnp.float32),
        pltpu.VMEM((_CH,), jnp.float32),
        pltpu.VMEM((_CH,), jnp.float32),
        pltpu.SemaphoreType.DMA,
        pltpu.SemaphoreType.DMA,
        pltpu.SemaphoreType.DMA,
    ],
)
def _cplr_sc(users, pos_items, neg_items, item_biases, ue, ie, out,
             iu, ip_, in_,
             ru0, rp0, rn0, ru1, rp1, rn1,
             pb0, nb0, pb1, nb1, out_v, sem0, sem1, isem):
    wid = lax.axis_index("s") * _NC + lax.axis_index("c")
    base = wid * _BPW
    lanes = lax.iota(jnp.int32, 16)

    row_sets = [(ru0, rp0, rn0), (ru1, rp1, rn1)]
    bias_sets = [(pb0, nb0), (pb1, nb1)]
    sems = [sem0, sem1]

    icopies = [
        pltpu.async_copy(users.at[pl.ds(base, _BPW)], iu, isem),
        pltpu.async_copy(pos_items.at[pl.ds(base, _BPW)], ip_, isem),
        pltpu.async_copy(neg_items.at[pl.ds(base, _BPW)], in_, isem),
    ]
    for cp in icopies:
        cp.wait()

    def stage_and_fire(c):
        s = c % 2
        ru, rp, rn = row_sets[s]
        pb, nb = bias_sets[s]
        coff = c * _CH
        iuc = iu.at[pl.ds(coff, _CH)]
        ipc = ip_.at[pl.ds(coff, _CH)]
        inc = in_.at[pl.ds(coff, _CH)]
        return [
            pltpu.async_copy(ue.at[iuc], ru, sems[s]),
            pltpu.async_copy(ie.at[ipc], rp, sems[s]),
            pltpu.async_copy(ie.at[inc], rn, sems[s]),
            pltpu.async_copy(item_biases.at[ipc], pb, sems[s]),
            pltpu.async_copy(item_biases.at[inc], nb, sems[s]),
        ]

    pending = {0: stage_and_fire(0)}
    for c in range(_NCHUNK):
        s = c % 2
        if c + 1 < _NCHUNK:
            pending[c + 1] = stage_and_fire(c + 1)
        for cp in pending.pop(c):
            cp.wait()
        ru, rp, rn = row_sets[s]
        pb, nb = bias_sets[s]
        off = base + c * _CH

        @plsc.parallel_loop(0, _CH // 16, step=1)
        def group_body(g, ru=ru, rp=rp, rn=rn, pb=pb, nb=nb):
            e0 = g * 16
            tot = jnp.zeros((16,), jnp.float32)
            for i in range(16):
                e = e0 + i
                prods = []
                for j in range(_D // 16):
                    u = ru[e, pl.ds(j * 16, 16)]
                    p = rp[e, pl.ds(j * 16, 16)]
                    n = rn[e, pl.ds(j * 16, 16)]
                    prods.append(u * (p - n))
                while len(prods) > 1:
                    prods = [prods[k] + prods[k + 1]
                             for k in range(0, len(prods), 2)]
                acc = prods[0]
                vals = [acc[k] for k in range(16)]
                while len(vals) > 1:
                    vals = [vals[k] + vals[k + 1]
                            for k in range(0, len(vals), 2)]
                tot = jnp.where(lanes == i, vals[0], tot)
            out_v[pl.ds(e0, 16)] = (tot + pb[pl.ds(e0, 16)]
                                    - nb[pl.ds(e0, 16)])

        pltpu.sync_copy(out_v, out.at[pl.ds(off, _CH)])


def kernel(users, pos_items, neg_items, user_biases, item_biases,
           user_embeddings, item_embeddings):
    del user_biases
    return _cplr_sc(
        users.astype(jnp.int32),
        pos_items.astype(jnp.int32),
        neg_items.astype(jnp.int32),
        item_biases.reshape(-1),
        user_embeddings,
        item_embeddings,
    )


# speedup vs baseline: 1.8773x; 1.0188x over previous
import functools

import jax
import jax.numpy as jnp
from jax import lax
from jax.experimental import pallas as pl
from jax.experimental.pallas import tpu as pltpu
from jax.experimental.pallas import tpu_sc as plsc

_B = 16384
_D = 128
_NC = 2
_NS = 16
_NW = _NC * _NS
_BPW = _B // _NW
_CH = 128
_NCHUNK = _BPW // _CH

_mesh = plsc.VectorSubcoreMesh(core_axis_name="c", subcore_axis_name="s")


@functools.partial(
    pl.kernel,
    mesh=_mesh,
    out_type=jax.ShapeDtypeStruct((_B,), jnp.float32),
    scratch_types=[
        pltpu.VMEM((_BPW,), jnp.int32),
        pltpu.VMEM((_BPW,), jnp.int32),
        pltpu.VMEM((_BPW,), jnp.int32),
        pltpu.VMEM((_CH, _D), jnp.float32),
        pltpu.VMEM((_CH, _D), jnp.float32),
        pltpu.VMEM((_CH, _D), jnp.float32),
        pltpu.VMEM((_CH, _D), jnp.float32),
        pltpu.VMEM((_CH, _D), jnp.float32),
        pltpu.VMEM((_CH, _D), jnp.float32),
        pltpu.VMEM((_CH,), jnp.float32),
        pltpu.VMEM((_CH,), jnp.float32),
        pltpu.VMEM((_CH,), jnp.float32),
        pltpu.VMEM((_CH,), jnp.float32),
        pltpu.VMEM((_CH,), jnp.float32),
        pltpu.SemaphoreType.DMA,
        pltpu.SemaphoreType.DMA,
        pltpu.SemaphoreType.DMA,
    ],
)
def _cplr_sc(users, pos_items, neg_items, item_biases, ue, ie, out,
             iu, ip_, in_,
             ru0, rp0, rn0, ru1, rp1, rn1,
             pb0, nb0, pb1, nb1, out_v, sem0, sem1, isem):
    wid = lax.axis_index("s") * _NC + lax.axis_index("c")
    base = wid * _BPW
    lanes = lax.iota(jnp.int32, 16)

    row_sets = [(ru0, rp0, rn0), (ru1, rp1, rn1)]
    bias_sets = [(pb0, nb0), (pb1, nb1)]
    sems = [sem0, sem1]

    icopies = [
        pltpu.async_copy(users.at[pl.ds(base, _BPW)], iu, isem),
        pltpu.async_copy(pos_items.at[pl.ds(base, _BPW)], ip_, isem),
        pltpu.async_copy(neg_items.at[pl.ds(base, _BPW)], in_, isem),
    ]
    for cp in icopies:
        cp.wait()

    def stage_and_fire(c):
        s = c % 2
        ru, rp, rn = row_sets[s]
        pb, nb = bias_sets[s]
        coff = c * _CH
        iuc = iu.at[pl.ds(coff, _CH)]
        ipc = ip_.at[pl.ds(coff, _CH)]
        inc = in_.at[pl.ds(coff, _CH)]
        return [
            pltpu.async_copy(ue.at[iuc], ru, sems[s]),
            pltpu.async_copy(ie.at[ipc], rp, sems[s]),
            pltpu.async_copy(ie.at[inc], rn, sems[s]),
            pltpu.async_copy(item_biases.at[ipc], pb, sems[s]),
            pltpu.async_copy(item_biases.at[inc], nb, sems[s]),
        ]

    pending = {0: stage_and_fire(0)}
    for c in range(_NCHUNK):
        s = c % 2
        if c + 1 < _NCHUNK:
            pending[c + 1] = stage_and_fire(c + 1)
        for cp in pending.pop(c):
            cp.wait()
        ru, rp, rn = row_sets[s]
        pb, nb = bias_sets[s]
        off = base + c * _CH

        @plsc.parallel_loop(0, _CH // 16, step=1)
        def group_body(g, ru=ru, rp=rp, rn=rn, pb=pb, nb=nb):
            e0 = g * 16
            tot = jnp.zeros((16,), jnp.float32)
            for i in range(16):
                e = e0 + i
                acc0 = None
                acc1 = None
                for j in range(_D // 16):
                    u = ru[e, pl.ds(j * 16, 16)]
                    p = rp[e, pl.ds(j * 16, 16)]
                    n = rn[e, pl.ds(j * 16, 16)]
                    d = u * (p - n)
                    if j % 2 == 0:
                        acc0 = d if acc0 is None else acc0 + d
                    else:
                        acc1 = d if acc1 is None else acc1 + d
                acc = acc0 + acc1
                vals = [acc[k] for k in range(16)]
                while len(vals) > 1:
                    vals = [vals[k] + vals[k + 1]
                            for k in range(0, len(vals), 2)]
                tot = jnp.where(lanes == i, vals[0], tot)
            out_v[pl.ds(e0, 16)] = (tot + pb[pl.ds(e0, 16)]
                                    - nb[pl.ds(e0, 16)])

        pltpu.sync_copy(out_v, out.at[pl.ds(off, _CH)])


def kernel(users, pos_items, neg_items, user_biases, item_biases,
           user_embeddings, item_embeddings):
    del user_biases
    return _cplr_sc(
        users.astype(jnp.int32),
        pos_items.astype(jnp.int32),
        neg_items.astype(jnp.int32),
        item_biases.reshape(-1),
        user_embeddings,
        item_embeddings,
    )
